# baseline (speedup 1.0000x reference)
import functools

import jax
import jax.numpy as jnp
from jax import lax
from jax.experimental import pallas as pl
from jax.experimental.pallas import tpu as pltpu
from jax.experimental.pallas import tpu_sc as plsc

N = 10000
E = 320000
EPS = 1e-5

NC = 2
NS = 16
NW = NC * NS
CHUNK = 128
EPT = 10240
CPT = EPT // CHUNK
EP = NW * EPT
EPAD = 2 * CHUNK
NP = 10112
RPT = NP // NS
DCH = 128
DCPT = EPT // DCH

def _stripe_chunks(step):
    out, off = [], 0
    while off < RPT:
        out.append((off, min(step, RPT - off)))
        off += step
    return tuple(out)


def _zero_vmem_2d(buf, nrows, ncols16):
    z = jnp.zeros((16,), jnp.float32)

    def body(i, c):
        for j in range(ncols16):
            buf[i, pl.ds(16 * j, 16)] = z
        return c

    lax.fori_loop(0, nrows, body, 0)


def _segsum_body(h_hbm, src_hbm, dst_hbm, out_hbm, acc, sidx0, sidx1,
                 didx0, didx1, rows0, rows1, si0, si1, sd0, sd1,
                 sg0, sg1):
    cid = lax.axis_index("c")
    sid = lax.axis_index("s")
    ebase = (cid * NS + sid) * EPT
    npairs = CPT // 2

    def ixs_start(t, buf, sem):
        pltpu.async_copy(src_hbm.at[pl.ds(ebase + t * CHUNK, CHUNK)],
                         buf, sem)

    def ixs_wait(t, buf, sem):
        pltpu.make_async_copy(src_hbm.at[pl.ds(ebase + t * CHUNK, CHUNK)],
                              buf, sem).wait()

    def ixd_start(t, buf, sem):
        pltpu.async_copy(dst_hbm.at[pl.ds(ebase + t * CHUNK, CHUNK)],
                         buf, sem)

    def ixd_wait(t, buf, sem):
        pltpu.make_async_copy(dst_hbm.at[pl.ds(ebase + t * CHUNK, CHUNK)],
                              buf, sem).wait()

    _zero_vmem_2d(rows0, CHUNK, 8)
    rbase = sid * RPT
    for off, n in _stripe_chunks(CHUNK):
        pltpu.sync_copy(rows0.at[pl.ds(0, n)], acc.at[pl.ds(rbase + off, n)])
    plsc.subcore_barrier()

    def body(i, c):
        t = 2 * i
        off0 = ebase + t * CHUNK
        off1 = ebase + (t + 1) * CHUNK
        pltpu.sync_copy(src_hbm.at[pl.ds(off0, CHUNK)], sidx0)
        pltpu.sync_copy(src_hbm.at[pl.ds(off1, CHUNK)], sidx1)
        pltpu.sync_copy(dst_hbm.at[pl.ds(off0, CHUNK)], didx0)
        pltpu.sync_copy(dst_hbm.at[pl.ds(off1, CHUNK)], didx1)
        g0 = pltpu.async_copy(h_hbm.at[sidx0], rows0, sg0)
        g1 = pltpu.async_copy(h_hbm.at[sidx1], rows1, sg1)
        g0.wait()
        g1.wait()
        pltpu.sync_copy(rows0, acc.at[didx0], add=True)
        pltpu.sync_copy(rows1, acc.at[didx1], add=True)
        return c

    lax.fori_loop(0, npairs, body, 0)
    plsc.subcore_barrier()
    pltpu.sync_copy(acc.at[pl.ds(rbase, RPT)],
                    out_hbm.at[cid, pl.ds(rbase, RPT)])


def _sc_segsum(h_pad, src_p, dst_p):
    mesh = plsc.VectorSubcoreMesh(core_axis_name="c", subcore_axis_name="s")
    return pl.kernel(
        _segsum_body,
        out_type=jax.ShapeDtypeStruct((NC, NP, 128), jnp.float32),
        mesh=mesh,
        scratch_types=[
            pltpu.VMEM_SHARED((NP, 128), jnp.float32),
            pltpu.VMEM((CHUNK,), jnp.int32),
            pltpu.VMEM((CHUNK,), jnp.int32),
            pltpu.VMEM((CHUNK,), jnp.int32),
            pltpu.VMEM((CHUNK,), jnp.int32),
            pltpu.VMEM((CHUNK, 128), jnp.float32),
            pltpu.VMEM((CHUNK, 128), jnp.float32),
            pltpu.SemaphoreType.DMA,
            pltpu.SemaphoreType.DMA,
            pltpu.SemaphoreType.DMA,
            pltpu.SemaphoreType.DMA,
            pltpu.SemaphoreType.DMA,
            pltpu.SemaphoreType.DMA,
        ],
    )(h_pad, src_p, dst_p)


def _deg_body(dst_hbm, out_hbm, acc, didx, ones, zbuf):
    cid = lax.axis_index("c")
    sid = lax.axis_index("s")

    _zero_vmem_2d(zbuf, DCH, 1)
    ov = jnp.ones((16,), jnp.float32)

    def fill(i, c):
        ones[i, :] = ov
        return c

    lax.fori_loop(0, DCH, fill, 0)

    rbase = sid * RPT
    for off, n in _stripe_chunks(DCH):
        pltpu.sync_copy(zbuf.at[pl.ds(0, n)], acc.at[pl.ds(rbase + off, n)])
    plsc.subcore_barrier()

    ebase = (cid * NS + sid) * EPT

    def body(t, c):
        pltpu.sync_copy(dst_hbm.at[pl.ds(ebase + t * DCH, DCH)], didx)
        pltpu.sync_copy(ones, acc.at[didx], add=True)
        return c

    lax.fori_loop(0, DCPT, body, 0)
    plsc.subcore_barrier()
    pltpu.sync_copy(acc.at[pl.ds(rbase, RPT)],
                    out_hbm.at[cid, pl.ds(rbase, RPT)])


def _sc_deg(dst_p):
    mesh = plsc.VectorSubcoreMesh(core_axis_name="c", subcore_axis_name="s")
    return pl.kernel(
        _deg_body,
        out_type=jax.ShapeDtypeStruct((NC, NP, 16), jnp.float32),
        mesh=mesh,
        scratch_types=[
            pltpu.VMEM_SHARED((NP, 16), jnp.float32),
            pltpu.VMEM((DCH,), jnp.int32),
            pltpu.VMEM((DCH, 16), jnp.float32),
            pltpu.VMEM((DCH, 16), jnp.float32),
        ],
    )(dst_p)


def _batch_norm(h, gamma, beta):
    m = jnp.mean(h, axis=0, keepdims=True)
    hc = h - m
    v = jnp.mean(hc * hc, axis=0, keepdims=True)
    return hc * lax.rsqrt(v + EPS) * gamma[None, :] + beta[None, :]


def _dinv_from_parts(degp_ref):
    degp = degp_ref[...]
    deg = degp[0, :, 0:1] + degp[1, :, 0:1] + 1.0
    return lax.rsqrt(deg)


def _tc1_body(x_ref, w_ref, g_ref, b_ref, degp_ref, hs_ref):
    x = x_ref[...]
    xn = _batch_norm(x, g_ref[...], b_ref[...])
    h1 = jnp.dot(xn, w_ref[...], preferred_element_type=jnp.float32)
    dinv = _dinv_from_parts(degp_ref)
    hs_ref[pl.ds(0, N), :] = h1 * dinv[0:N]
    hs_ref[pl.ds(N, NP - N), :] = jnp.zeros((NP - N, 128), jnp.float32)


def _tc1(x, w1, g0, b0, degp):
    return pl.pallas_call(
        _tc1_body,
        out_shape=jax.ShapeDtypeStruct((NP, 128), jnp.float32),
    )(x, w1, g0, b0, degp)


def _tc2_body(segp_ref, hs_ref, degp_ref, b_ref, g1_ref, be1_ref, out_ref):
    segp = segp_ref[...]
    seg = segp[0, 0:N, :] + segp[1, 0:N, :]
    dinv = _dinv_from_parts(degp_ref)[0:N]
    hs = hs_ref[pl.ds(0, N), :]
    h = jax.nn.relu(dinv * (seg + hs) + b_ref[...][None, :])
    out_ref[pl.ds(0, N), :] = _batch_norm(h, g1_ref[...], be1_ref[...])
    out_ref[pl.ds(N, NP - N), :] = jnp.zeros((NP - N, 128), jnp.float32)


def _tc2(segp, hs, degp, b1, g1, be1):
    return pl.pallas_call(
        _tc2_body,
        out_shape=jax.ShapeDtypeStruct((NP, 128), jnp.float32),
    )(segp, hs, degp, b1, g1, be1)


def _tc_graph_body(aggp_ref, x_ref, wrel_ref, wroot_ref, b_ref, g_ref,
                   be_ref, out_ref, *, hout, pad_out):
    aggp = aggp_ref[...]
    agg = aggp[0, 0:N, :] + aggp[1, 0:N, :]
    x = x_ref[pl.ds(0, N), :]
    y = (jnp.dot(agg, wrel_ref[...], preferred_element_type=jnp.float32)
         + jnp.dot(x, wroot_ref[...], preferred_element_type=jnp.float32)
         + b_ref[...][None, :])
    h = _batch_norm(jax.nn.relu(y), g_ref[...], be_ref[...])
    if pad_out:
        out_ref[pl.ds(0, N), :] = h
        out_ref[pl.ds(N, NP - N), :] = jnp.zeros((NP - N, hout), jnp.float32)
    else:
        out_ref[...] = h


def _tc_graph(aggp, x, wrel, wroot, b, g, be, hout, pad_out):
    nrows = NP if pad_out else N
    return pl.pallas_call(
        functools.partial(_tc_graph_body, hout=hout, pad_out=pad_out),
        out_shape=jax.ShapeDtypeStruct((nrows, hout), jnp.float32),
    )(aggp, x, wrel, wroot, b, g, be)


def kernel(x, edge_index, gamma0, beta0, W1, b1, gamma1, beta1, Wrel2,
           Wroot2, b2, gamma2, beta2, Wrel3, Wroot3, b3, gamma3, beta3,
           Wrel4, Wroot4, b4, gamma4, beta4):
    pad = jnp.full((EP - E + EPAD,), N, dtype=jnp.int32)
    src_p = jnp.concatenate([edge_index[0], pad])
    dst_p = jnp.concatenate([edge_index[1], pad])

    degp = _sc_deg(dst_p)
    hs = _tc1(x, W1, gamma0, beta0, degp)
    segp = _sc_segsum(hs, src_p, dst_p)
    x2 = _tc2(segp, hs, degp, b1, gamma1, beta1)
    aggp = _sc_segsum(x2, src_p, dst_p)
    x3 = _tc_graph(aggp, x2, Wrel2, Wroot2, b2, gamma2, beta2, 128, True)
    aggp = _sc_segsum(x3, src_p, dst_p)
    x4 = _tc_graph(aggp, x3, Wrel3, Wroot3, b3, gamma3, beta3, 128, True)
    aggp = _sc_segsum(x4, src_p, dst_p)
    return _tc_graph(aggp, x4, Wrel4, Wroot4, b4, gamma4, beta4, 64, False)


# speedup vs baseline: 1.0270x; 1.0270x over previous
import functools

import jax
import jax.numpy as jnp
from jax import lax
from jax.experimental import pallas as pl
from jax.experimental.pallas import tpu as pltpu
from jax.experimental.pallas import tpu_sc as plsc

N = 10000
E = 320000
EPS = 1e-5

NC = 2
NS = 16
NW = NC * NS
CHUNK = 128
EPT = 10240
CPT = EPT // CHUNK
EP = NW * EPT
EPAD = 2 * CHUNK
NP = 10112
RPT = NP // NS
DCH = 128
DCPT = EPT // DCH
GRP = 8

def _stripe_chunks(step):
    out, off = [], 0
    while off < RPT:
        out.append((off, min(step, RPT - off)))
        off += step
    return tuple(out)


def _zero_vmem_2d(buf, nrows, ncols16):
    z = jnp.zeros((16,), jnp.float32)

    def body(i, c):
        for j in range(ncols16):
            buf[i, pl.ds(16 * j, 16)] = z
        return c

    lax.fori_loop(0, nrows, body, 0)


def _segsum_body(h_hbm, src_hbm, dst_hbm, out_hbm, acc, sidxb, didxb,
                 rows0, rows1, sg0, sg1, ss0, ss1):
    cid = lax.axis_index("c")
    sid = lax.axis_index("s")
    wid = cid * NS + sid

    _zero_vmem_2d(rows0, CHUNK, 8)
    rbase = sid * RPT
    for off, n in _stripe_chunks(CHUNK):
        pltpu.sync_copy(rows0.at[pl.ds(0, n)], acc.at[pl.ds(rbase + off, n)])
    plsc.subcore_barrier()

    def body(j, c):
        crow = wid * CPT + j * GRP
        pltpu.sync_copy(src_hbm.at[pl.ds(crow, GRP)], sidxb)
        pltpu.sync_copy(dst_hbm.at[pl.ds(crow, GRP)], didxb)
        for k in range(0, GRP, 2):
            g0 = pltpu.async_copy(h_hbm.at[sidxb.at[k]], rows0, sg0)
            g1 = pltpu.async_copy(h_hbm.at[sidxb.at[k + 1]], rows1, sg1)
            g0.wait()
            g1.wait()
            pltpu.sync_copy(rows0, acc.at[didxb.at[k]], add=True)
            pltpu.sync_copy(rows1, acc.at[didxb.at[k + 1]], add=True)
        return c

    lax.fori_loop(0, CPT // GRP, body, 0)
    plsc.subcore_barrier()
    pltpu.sync_copy(acc.at[pl.ds(rbase, RPT)],
                    out_hbm.at[cid, pl.ds(rbase, RPT)])


def _sc_segsum(h_pad, src_p, dst_p):
    mesh = plsc.VectorSubcoreMesh(core_axis_name="c", subcore_axis_name="s")
    return pl.kernel(
        _segsum_body,
        out_type=jax.ShapeDtypeStruct((NC, NP, 128), jnp.float32),
        mesh=mesh,
        scratch_types=[
            pltpu.VMEM_SHARED((NP, 128), jnp.float32),
            pltpu.VMEM((GRP, CHUNK), jnp.int32),
            pltpu.VMEM((GRP, CHUNK), jnp.int32),
            pltpu.VMEM((CHUNK, 128), jnp.float32),
            pltpu.VMEM((CHUNK, 128), jnp.float32),
            pltpu.SemaphoreType.DMA,
            pltpu.SemaphoreType.DMA,
            pltpu.SemaphoreType.DMA,
            pltpu.SemaphoreType.DMA,
        ],
    )(h_pad, src_p, dst_p)


def _deg_body(dst_hbm, out_hbm, acc, didx, ones, zbuf):
    cid = lax.axis_index("c")
    sid = lax.axis_index("s")

    _zero_vmem_2d(zbuf, DCH, 1)
    ov = jnp.ones((16,), jnp.float32)

    def fill(i, c):
        ones[i, :] = ov
        return c

    lax.fori_loop(0, DCH, fill, 0)

    rbase = sid * RPT
    for off, n in _stripe_chunks(DCH):
        pltpu.sync_copy(zbuf.at[pl.ds(0, n)], acc.at[pl.ds(rbase + off, n)])
    plsc.subcore_barrier()

    ebase = (cid * NS + sid) * EPT

    def body(t, c):
        pltpu.sync_copy(dst_hbm.at[pl.ds(ebase + t * DCH, DCH)], didx)
        pltpu.sync_copy(ones, acc.at[didx], add=True)
        return c

    lax.fori_loop(0, DCPT, body, 0)
    plsc.subcore_barrier()
    pltpu.sync_copy(acc.at[pl.ds(rbase, RPT)],
                    out_hbm.at[cid, pl.ds(rbase, RPT)])


def _sc_deg(dst_p):
    mesh = plsc.VectorSubcoreMesh(core_axis_name="c", subcore_axis_name="s")
    return pl.kernel(
        _deg_body,
        out_type=jax.ShapeDtypeStruct((NC, NP, 16), jnp.float32),
        mesh=mesh,
        scratch_types=[
            pltpu.VMEM_SHARED((NP, 16), jnp.float32),
            pltpu.VMEM((DCH,), jnp.int32),
            pltpu.VMEM((DCH, 16), jnp.float32),
            pltpu.VMEM((DCH, 16), jnp.float32),
        ],
    )(dst_p)


def _batch_norm(h, gamma, beta):
    m = jnp.mean(h, axis=0, keepdims=True)
    hc = h - m
    v = jnp.mean(hc * hc, axis=0, keepdims=True)
    return hc * lax.rsqrt(v + EPS) * gamma[None, :] + beta[None, :]


def _dinv_from_parts(degp_ref):
    degp = degp_ref[...]
    deg = degp[0, :, 0:1] + degp[1, :, 0:1] + 1.0
    return lax.rsqrt(deg)


def _tc1_body(x_ref, w_ref, g_ref, b_ref, degp_ref, hs_ref):
    x = x_ref[...]
    xn = _batch_norm(x, g_ref[...], b_ref[...])
    h1 = jnp.dot(xn, w_ref[...], preferred_element_type=jnp.float32)
    dinv = _dinv_from_parts(degp_ref)
    hs_ref[pl.ds(0, N), :] = h1 * dinv[0:N]
    hs_ref[pl.ds(N, NP - N), :] = jnp.zeros((NP - N, 128), jnp.float32)


def _tc1(x, w1, g0, b0, degp):
    return pl.pallas_call(
        _tc1_body,
        out_shape=jax.ShapeDtypeStruct((NP, 128), jnp.float32),
    )(x, w1, g0, b0, degp)


def _tc2_body(segp_ref, hs_ref, degp_ref, b_ref, g1_ref, be1_ref, out_ref):
    segp = segp_ref[...]
    seg = segp[0, 0:N, :] + segp[1, 0:N, :]
    dinv = _dinv_from_parts(degp_ref)[0:N]
    hs = hs_ref[pl.ds(0, N), :]
    h = jax.nn.relu(dinv * (seg + hs) + b_ref[...][None, :])
    out_ref[pl.ds(0, N), :] = _batch_norm(h, g1_ref[...], be1_ref[...])
    out_ref[pl.ds(N, NP - N), :] = jnp.zeros((NP - N, 128), jnp.float32)


def _tc2(segp, hs, degp, b1, g1, be1):
    return pl.pallas_call(
        _tc2_body,
        out_shape=jax.ShapeDtypeStruct((NP, 128), jnp.float32),
    )(segp, hs, degp, b1, g1, be1)


def _tc_graph_body(aggp_ref, x_ref, wrel_ref, wroot_ref, b_ref, g_ref,
                   be_ref, out_ref, *, hout, pad_out):
    aggp = aggp_ref[...]
    agg = aggp[0, 0:N, :] + aggp[1, 0:N, :]
    x = x_ref[pl.ds(0, N), :]
    y = (jnp.dot(agg, wrel_ref[...], preferred_element_type=jnp.float32)
         + jnp.dot(x, wroot_ref[...], preferred_element_type=jnp.float32)
         + b_ref[...][None, :])
    h = _batch_norm(jax.nn.relu(y), g_ref[...], be_ref[...])
    if pad_out:
        out_ref[pl.ds(0, N), :] = h
        out_ref[pl.ds(N, NP - N), :] = jnp.zeros((NP - N, hout), jnp.float32)
    else:
        out_ref[...] = h


def _tc_graph(aggp, x, wrel, wroot, b, g, be, hout, pad_out):
    nrows = NP if pad_out else N
    return pl.pallas_call(
        functools.partial(_tc_graph_body, hout=hout, pad_out=pad_out),
        out_shape=jax.ShapeDtypeStruct((nrows, hout), jnp.float32),
    )(aggp, x, wrel, wroot, b, g, be)


def kernel(x, edge_index, gamma0, beta0, W1, b1, gamma1, beta1, Wrel2,
           Wroot2, b2, gamma2, beta2, Wrel3, Wroot3, b3, gamma3, beta3,
           Wrel4, Wroot4, b4, gamma4, beta4):
    pad = jnp.full((EP - E,), N, dtype=jnp.int32)
    src_p = jnp.concatenate([edge_index[0], pad])
    dst_p = jnp.concatenate([edge_index[1], pad])
    src_2d = src_p.reshape(EP // CHUNK, CHUNK)
    dst_2d = dst_p.reshape(EP // CHUNK, CHUNK)

    degp = _sc_deg(dst_p)
    hs = _tc1(x, W1, gamma0, beta0, degp)
    segp = _sc_segsum(hs, src_2d, dst_2d)
    x2 = _tc2(segp, hs, degp, b1, gamma1, beta1)
    aggp = _sc_segsum(x2, src_2d, dst_2d)
    x3 = _tc_graph(aggp, x2, Wrel2, Wroot2, b2, gamma2, beta2, 128, True)
    aggp = _sc_segsum(x3, src_2d, dst_2d)
    x4 = _tc_graph(aggp, x3, Wrel3, Wroot3, b3, gamma3, beta3, 128, True)
    aggp = _sc_segsum(x4, src_2d, dst_2d)
    return _tc_graph(aggp, x4, Wrel4, Wroot4, b4, gamma4, beta4, 64, False)


# speedup vs baseline: 1.1037x; 1.0747x over previous
import functools

import jax
import jax.numpy as jnp
from jax import lax
from jax.experimental import pallas as pl
from jax.experimental.pallas import tpu as pltpu
from jax.experimental.pallas import tpu_sc as plsc

N = 10000
E = 320000
EPS = 1e-5

NC = 2
NS = 16
NW = NC * NS
CHUNK = 128
EPT = 10240
CPT = EPT // CHUNK
EP = NW * EPT
EPAD = 2 * CHUNK
NP = 10112
RPT = NP // NS
DCH = 128
DCPT = EPT // DCH
GRP = 8

def _stripe_chunks(step):
    out, off = [], 0
    while off < RPT:
        out.append((off, min(step, RPT - off)))
        off += step
    return tuple(out)


def _zero_vmem_2d(buf, nrows, ncols16):
    z = jnp.zeros((16,), jnp.float32)

    def body(i, c):
        for j in range(ncols16):
            buf[i, pl.ds(16 * j, 16)] = z
        return c

    lax.fori_loop(0, nrows, body, 0)


def _segsum_body(h_hbm, src_hbm, dst_hbm, out_hbm, acc, sidxb, didxb,
                 rows0, rows1, sg0, sg1, ss0, ss1):
    cid = lax.axis_index("c")
    sid = lax.axis_index("s")
    wid = cid * NS + sid

    _zero_vmem_2d(rows0, CHUNK, 8)
    rbase = sid * RPT
    for off, n in _stripe_chunks(CHUNK):
        pltpu.sync_copy(rows0.at[pl.ds(0, n)], acc.at[pl.ds(rbase + off, n)])
    plsc.subcore_barrier()

    def body(j, c):
        crow = wid * CPT + j * GRP
        pltpu.sync_copy(src_hbm.at[pl.ds(crow, GRP)], sidxb)
        pltpu.sync_copy(dst_hbm.at[pl.ds(crow, GRP)], didxb)
        bufs = (rows0, rows1)
        sems = (sg0, sg1)
        g = pltpu.async_copy(h_hbm.at[sidxb.at[0]], rows0, sg0)
        for k in range(GRP):
            nxt = None
            if k + 1 < GRP:
                nxt = pltpu.async_copy(h_hbm.at[sidxb.at[k + 1]],
                                       bufs[(k + 1) % 2], sems[(k + 1) % 2])
            g.wait()
            pltpu.sync_copy(bufs[k % 2], acc.at[didxb.at[k]], add=True)
            g = nxt
        return c

    lax.fori_loop(0, CPT // GRP, body, 0)
    plsc.subcore_barrier()
    pltpu.sync_copy(acc.at[pl.ds(rbase, RPT)],
                    out_hbm.at[cid, pl.ds(rbase, RPT)])


def _sc_segsum(h_pad, src_p, dst_p):
    mesh = plsc.VectorSubcoreMesh(core_axis_name="c", subcore_axis_name="s")
    return pl.kernel(
        _segsum_body,
        out_type=jax.ShapeDtypeStruct((NC, NP, 128), jnp.float32),
        mesh=mesh,
        scratch_types=[
            pltpu.VMEM_SHARED((NP, 128), jnp.float32),
            pltpu.VMEM((GRP, CHUNK), jnp.int32),
            pltpu.VMEM((GRP, CHUNK), jnp.int32),
            pltpu.VMEM((CHUNK, 128), jnp.float32),
            pltpu.VMEM((CHUNK, 128), jnp.float32),
            pltpu.SemaphoreType.DMA,
            pltpu.SemaphoreType.DMA,
            pltpu.SemaphoreType.DMA,
            pltpu.SemaphoreType.DMA,
        ],
    )(h_pad, src_p, dst_p)


def _deg_body(dst_hbm, out_hbm, acc, didx, ones, zbuf):
    cid = lax.axis_index("c")
    sid = lax.axis_index("s")

    _zero_vmem_2d(zbuf, DCH, 1)
    ov = jnp.ones((16,), jnp.float32)

    def fill(i, c):
        ones[i, :] = ov
        return c

    lax.fori_loop(0, DCH, fill, 0)

    rbase = sid * RPT
    for off, n in _stripe_chunks(DCH):
        pltpu.sync_copy(zbuf.at[pl.ds(0, n)], acc.at[pl.ds(rbase + off, n)])
    plsc.subcore_barrier()

    ebase = (cid * NS + sid) * EPT

    def body(t, c):
        pltpu.sync_copy(dst_hbm.at[pl.ds(ebase + t * DCH, DCH)], didx)
        pltpu.sync_copy(ones, acc.at[didx], add=True)
        return c

    lax.fori_loop(0, DCPT, body, 0)
    plsc.subcore_barrier()
    pltpu.sync_copy(acc.at[pl.ds(rbase, RPT)],
                    out_hbm.at[cid, pl.ds(rbase, RPT)])


def _sc_deg(dst_p):
    mesh = plsc.VectorSubcoreMesh(core_axis_name="c", subcore_axis_name="s")
    return pl.kernel(
        _deg_body,
        out_type=jax.ShapeDtypeStruct((NC, NP, 16), jnp.float32),
        mesh=mesh,
        scratch_types=[
            pltpu.VMEM_SHARED((NP, 16), jnp.float32),
            pltpu.VMEM((DCH,), jnp.int32),
            pltpu.VMEM((DCH, 16), jnp.float32),
            pltpu.VMEM((DCH, 16), jnp.float32),
        ],
    )(dst_p)


def _batch_norm(h, gamma, beta):
    m = jnp.mean(h, axis=0, keepdims=True)
    hc = h - m
    v = jnp.mean(hc * hc, axis=0, keepdims=True)
    return hc * lax.rsqrt(v + EPS) * gamma[None, :] + beta[None, :]


def _dinv_from_parts(degp_ref):
    degp = degp_ref[...]
    deg = degp[0, :, 0:1] + degp[1, :, 0:1] + 1.0
    return lax.rsqrt(deg)


def _tc1_body(x_ref, w_ref, g_ref, b_ref, degp_ref, hs_ref):
    x = x_ref[...]
    xn = _batch_norm(x, g_ref[...], b_ref[...])
    h1 = jnp.dot(xn, w_ref[...], preferred_element_type=jnp.float32)
    dinv = _dinv_from_parts(degp_ref)
    hs_ref[pl.ds(0, N), :] = h1 * dinv[0:N]
    hs_ref[pl.ds(N, NP - N), :] = jnp.zeros((NP - N, 128), jnp.float32)


def _tc1(x, w1, g0, b0, degp):
    return pl.pallas_call(
        _tc1_body,
        out_shape=jax.ShapeDtypeStruct((NP, 128), jnp.float32),
    )(x, w1, g0, b0, degp)


def _tc2_body(segp_ref, hs_ref, degp_ref, b_ref, g1_ref, be1_ref, out_ref):
    segp = segp_ref[...]
    seg = segp[0, 0:N, :] + segp[1, 0:N, :]
    dinv = _dinv_from_parts(degp_ref)[0:N]
    hs = hs_ref[pl.ds(0, N), :]
    h = jax.nn.relu(dinv * (seg + hs) + b_ref[...][None, :])
    out_ref[pl.ds(0, N), :] = _batch_norm(h, g1_ref[...], be1_ref[...])
    out_ref[pl.ds(N, NP - N), :] = jnp.zeros((NP - N, 128), jnp.float32)


def _tc2(segp, hs, degp, b1, g1, be1):
    return pl.pallas_call(
        _tc2_body,
        out_shape=jax.ShapeDtypeStruct((NP, 128), jnp.float32),
    )(segp, hs, degp, b1, g1, be1)


def _tc_graph_body(aggp_ref, x_ref, wrel_ref, wroot_ref, b_ref, g_ref,
                   be_ref, out_ref, *, hout, pad_out):
    aggp = aggp_ref[...]
    agg = aggp[0, 0:N, :] + aggp[1, 0:N, :]
    x = x_ref[pl.ds(0, N), :]
    y = (jnp.dot(agg, wrel_ref[...], preferred_element_type=jnp.float32)
         + jnp.dot(x, wroot_ref[...], preferred_element_type=jnp.float32)
         + b_ref[...][None, :])
    h = _batch_norm(jax.nn.relu(y), g_ref[...], be_ref[...])
    if pad_out:
        out_ref[pl.ds(0, N), :] = h
        out_ref[pl.ds(N, NP - N), :] = jnp.zeros((NP - N, hout), jnp.float32)
    else:
        out_ref[...] = h


def _tc_graph(aggp, x, wrel, wroot, b, g, be, hout, pad_out):
    nrows = NP if pad_out else N
    return pl.pallas_call(
        functools.partial(_tc_graph_body, hout=hout, pad_out=pad_out),
        out_shape=jax.ShapeDtypeStruct((nrows, hout), jnp.float32),
    )(aggp, x, wrel, wroot, b, g, be)


def kernel(x, edge_index, gamma0, beta0, W1, b1, gamma1, beta1, Wrel2,
           Wroot2, b2, gamma2, beta2, Wrel3, Wroot3, b3, gamma3, beta3,
           Wrel4, Wroot4, b4, gamma4, beta4):
    pad = jnp.full((EP - E,), N, dtype=jnp.int32)
    src_p = jnp.concatenate([edge_index[0], pad])
    dst_p = jnp.concatenate([edge_index[1], pad])
    src_2d = src_p.reshape(EP // CHUNK, CHUNK)
    dst_2d = dst_p.reshape(EP // CHUNK, CHUNK)

    degp = _sc_deg(dst_p)
    hs = _tc1(x, W1, gamma0, beta0, degp)
    segp = _sc_segsum(hs, src_2d, dst_2d)
    x2 = _tc2(segp, hs, degp, b1, gamma1, beta1)
    aggp = _sc_segsum(x2, src_2d, dst_2d)
    x3 = _tc_graph(aggp, x2, Wrel2, Wroot2, b2, gamma2, beta2, 128, True)
    aggp = _sc_segsum(x3, src_2d, dst_2d)
    x4 = _tc_graph(aggp, x3, Wrel3, Wroot3, b3, gamma3, beta3, 128, True)
    aggp = _sc_segsum(x4, src_2d, dst_2d)
    return _tc_graph(aggp, x4, Wrel4, Wroot4, b4, gamma4, beta4, 64, False)


# speedup vs baseline: 2.3585x; 2.1369x over previous
import functools

import jax
import jax.numpy as jnp
from jax import lax
from jax.experimental import pallas as pl
from jax.experimental.pallas import tpu as pltpu
from jax.experimental.pallas import tpu_sc as plsc

N = 10000
E = 320000
EPS = 1e-5

NC = 2
NS = 16
NW = NC * NS
CHUNK = 128
EPT = 10240
CPT = EPT // CHUNK
EP = NW * EPT
EPAD = 2 * CHUNK
NP = 10112
RPT = NP // NS
DCH = 128
DCPT = EPT // DCH
GRP = 8

def _stripe_chunks(step):
    out, off = [], 0
    while off < RPT:
        out.append((off, min(step, RPT - off)))
        off += step
    return tuple(out)


def _zero_vmem_2d(buf, nrows, ncols16):
    z = jnp.zeros((16,), jnp.float32)

    def body(i, c):
        for j in range(ncols16):
            buf[i, pl.ds(16 * j, 16)] = z
        return c

    lax.fori_loop(0, nrows, body, 0)


HC = 64
TCPT = EP // CHUNK // NS


def _segsum_body(h0_hbm, h1_hbm, src_hbm, dst_hbm, out0_hbm, out1_hbm,
                 hsh, acc, sidxb, didxb, rows0, rows1, sg0, sg1):
    cid = lax.axis_index("c")
    sid = lax.axis_index("s")
    rbase = sid * RPT

    @pl.when(cid == 0)
    def _stage0():
        pltpu.sync_copy(h0_hbm.at[pl.ds(rbase, RPT)],
                        hsh.at[pl.ds(rbase, RPT)])

    @pl.when(cid == 1)
    def _stage1():
        pltpu.sync_copy(h1_hbm.at[pl.ds(rbase, RPT)],
                        hsh.at[pl.ds(rbase, RPT)])

    _zero_vmem_2d(rows0, CHUNK, HC // 16)
    for off, n in _stripe_chunks(CHUNK):
        pltpu.sync_copy(rows0.at[pl.ds(0, n)], acc.at[pl.ds(rbase + off, n)])
    plsc.subcore_barrier()

    def body(j, c):
        crow = sid * TCPT + j * GRP
        pltpu.sync_copy(src_hbm.at[pl.ds(crow, GRP)], sidxb)
        pltpu.sync_copy(dst_hbm.at[pl.ds(crow, GRP)], didxb)
        bufs = (rows0, rows1)
        sems = (sg0, sg1)
        g = pltpu.async_copy(hsh.at[sidxb.at[0]], rows0, sg0)
        for k in range(GRP):
            nxt = None
            if k + 1 < GRP:
                nxt = pltpu.async_copy(hsh.at[sidxb.at[k + 1]],
                                       bufs[(k + 1) % 2], sems[(k + 1) % 2])
            g.wait()
            pltpu.sync_copy(bufs[k % 2], acc.at[didxb.at[k]], add=True)
            g = nxt
        return c

    lax.fori_loop(0, TCPT // GRP, body, 0)
    plsc.subcore_barrier()

    @pl.when(cid == 0)
    def _out0():
        pltpu.sync_copy(acc.at[pl.ds(rbase, RPT)],
                        out0_hbm.at[pl.ds(rbase, RPT)])

    @pl.when(cid == 1)
    def _out1():
        pltpu.sync_copy(acc.at[pl.ds(rbase, RPT)],
                        out1_hbm.at[pl.ds(rbase, RPT)])


def _sc_segsum(h0, h1, src_p, dst_p):
    mesh = plsc.VectorSubcoreMesh(core_axis_name="c", subcore_axis_name="s")
    return pl.kernel(
        _segsum_body,
        out_type=(jax.ShapeDtypeStruct((NP, HC), jnp.float32),
                  jax.ShapeDtypeStruct((NP, HC), jnp.float32)),
        mesh=mesh,
        scratch_types=[
            pltpu.VMEM_SHARED((NP, HC), jnp.float32),
            pltpu.VMEM_SHARED((NP, HC), jnp.float32),
            pltpu.VMEM((GRP, CHUNK), jnp.int32),
            pltpu.VMEM((GRP, CHUNK), jnp.int32),
            pltpu.VMEM((CHUNK, HC), jnp.float32),
            pltpu.VMEM((CHUNK, HC), jnp.float32),
            pltpu.SemaphoreType.DMA,
            pltpu.SemaphoreType.DMA,
        ],
    )(h0, h1, src_p, dst_p)


def _deg_body(dst_hbm, out_hbm, acc, didx, ones, zbuf):
    cid = lax.axis_index("c")
    sid = lax.axis_index("s")

    _zero_vmem_2d(zbuf, DCH, 1)
    ov = jnp.ones((16,), jnp.float32)

    def fill(i, c):
        ones[i, :] = ov
        return c

    lax.fori_loop(0, DCH, fill, 0)

    rbase = sid * RPT
    for off, n in _stripe_chunks(DCH):
        pltpu.sync_copy(zbuf.at[pl.ds(0, n)], acc.at[pl.ds(rbase + off, n)])
    plsc.subcore_barrier()

    ebase = (cid * NS + sid) * EPT

    def body(t, c):
        pltpu.sync_copy(dst_hbm.at[pl.ds(ebase + t * DCH, DCH)], didx)
        pltpu.sync_copy(ones, acc.at[didx], add=True)
        return c

    lax.fori_loop(0, DCPT, body, 0)
    plsc.subcore_barrier()
    pltpu.sync_copy(acc.at[pl.ds(rbase, RPT)],
                    out_hbm.at[cid, pl.ds(rbase, RPT)])


def _sc_deg(dst_p):
    mesh = plsc.VectorSubcoreMesh(core_axis_name="c", subcore_axis_name="s")
    return pl.kernel(
        _deg_body,
        out_type=jax.ShapeDtypeStruct((NC, NP, 16), jnp.float32),
        mesh=mesh,
        scratch_types=[
            pltpu.VMEM_SHARED((NP, 16), jnp.float32),
            pltpu.VMEM((DCH,), jnp.int32),
            pltpu.VMEM((DCH, 16), jnp.float32),
            pltpu.VMEM((DCH, 16), jnp.float32),
        ],
    )(dst_p)


def _batch_norm(h, gamma, beta):
    m = jnp.mean(h, axis=0, keepdims=True)
    hc = h - m
    v = jnp.mean(hc * hc, axis=0, keepdims=True)
    return hc * lax.rsqrt(v + EPS) * gamma[None, :] + beta[None, :]


def _dinv_from_parts(degp_ref):
    degp = degp_ref[...]
    deg = degp[0, :, 0:1] + degp[1, :, 0:1] + 1.0
    return lax.rsqrt(deg)


def _store_split(ref0, ref1, mat):
    zpad = jnp.zeros((NP - N, HC), jnp.float32)
    ref0[pl.ds(0, N), :] = mat[:, 0:HC]
    ref0[pl.ds(N, NP - N), :] = zpad
    ref1[pl.ds(0, N), :] = mat[:, HC:128]
    ref1[pl.ds(N, NP - N), :] = zpad


def _tc1_body(x_ref, w_ref, g_ref, b_ref, degp_ref, hs0_ref, hs1_ref):
    x = x_ref[...]
    xn = _batch_norm(x, g_ref[...], b_ref[...])
    h1 = jnp.dot(xn, w_ref[...], preferred_element_type=jnp.float32)
    dinv = _dinv_from_parts(degp_ref)
    _store_split(hs0_ref, hs1_ref, h1 * dinv[0:N])


def _tc1(x, w1, g0, b0, degp):
    return pl.pallas_call(
        _tc1_body,
        out_shape=(jax.ShapeDtypeStruct((NP, HC), jnp.float32),
                   jax.ShapeDtypeStruct((NP, HC), jnp.float32)),
    )(x, w1, g0, b0, degp)


def _tc2_body(seg0_ref, seg1_ref, hs0_ref, hs1_ref, degp_ref, b_ref,
              g1_ref, be1_ref, out0_ref, out1_ref):
    dinv = _dinv_from_parts(degp_ref)[0:N]
    seg = jnp.concatenate(
        [seg0_ref[pl.ds(0, N), :], seg1_ref[pl.ds(0, N), :]], axis=1)
    hs = jnp.concatenate(
        [hs0_ref[pl.ds(0, N), :], hs1_ref[pl.ds(0, N), :]], axis=1)
    h = jax.nn.relu(dinv * (seg + hs) + b_ref[...][None, :])
    _store_split(out0_ref, out1_ref,
                 _batch_norm(h, g1_ref[...], be1_ref[...]))


def _tc2(seg0, seg1, hs0, hs1, degp, b1, g1, be1):
    return pl.pallas_call(
        _tc2_body,
        out_shape=(jax.ShapeDtypeStruct((NP, HC), jnp.float32),
                   jax.ShapeDtypeStruct((NP, HC), jnp.float32)),
    )(seg0, seg1, hs0, hs1, degp, b1, g1, be1)


def _tc_graph_body(agg0_ref, agg1_ref, x0_ref, x1_ref, wrel_ref, wroot_ref,
                   b_ref, g_ref, be_ref, *out_refs, hout):
    wrel = wrel_ref[...]
    wroot = wroot_ref[...]
    y = (jnp.dot(agg0_ref[pl.ds(0, N), :], wrel[0:HC],
                 preferred_element_type=jnp.float32)
         + jnp.dot(agg1_ref[pl.ds(0, N), :], wrel[HC:128],
                   preferred_element_type=jnp.float32)
         + jnp.dot(x0_ref[pl.ds(0, N), :], wroot[0:HC],
                   preferred_element_type=jnp.float32)
         + jnp.dot(x1_ref[pl.ds(0, N), :], wroot[HC:128],
                   preferred_element_type=jnp.float32)
         + b_ref[...][None, :])
    h = _batch_norm(jax.nn.relu(y), g_ref[...], be_ref[...])
    if hout == 128:
        _store_split(out_refs[0], out_refs[1], h)
    else:
        out_refs[0][...] = h


def _tc_graph(agg0, agg1, x0, x1, wrel, wroot, b, g, be, hout):
    if hout == 128:
        out_shape = (jax.ShapeDtypeStruct((NP, HC), jnp.float32),
                     jax.ShapeDtypeStruct((NP, HC), jnp.float32))
    else:
        out_shape = jax.ShapeDtypeStruct((N, hout), jnp.float32)
    return pl.pallas_call(
        functools.partial(_tc_graph_body, hout=hout),
        out_shape=out_shape,
    )(agg0, agg1, x0, x1, wrel, wroot, b, g, be)


def kernel(x, edge_index, gamma0, beta0, W1, b1, gamma1, beta1, Wrel2,
           Wroot2, b2, gamma2, beta2, Wrel3, Wroot3, b3, gamma3, beta3,
           Wrel4, Wroot4, b4, gamma4, beta4):
    pad = jnp.full((EP - E,), N, dtype=jnp.int32)
    src_p = jnp.concatenate([edge_index[0], pad])
    dst_p = jnp.concatenate([edge_index[1], pad])
    src_2d = src_p.reshape(EP // CHUNK, CHUNK)
    dst_2d = dst_p.reshape(EP // CHUNK, CHUNK)

    degp = _sc_deg(dst_p)
    hs0, hs1 = _tc1(x, W1, gamma0, beta0, degp)
    seg0, seg1 = _sc_segsum(hs0, hs1, src_2d, dst_2d)
    x2 = _tc2(seg0, seg1, hs0, hs1, degp, b1, gamma1, beta1)
    agg = _sc_segsum(x2[0], x2[1], src_2d, dst_2d)
    x3 = _tc_graph(agg[0], agg[1], x2[0], x2[1], Wrel2, Wroot2, b2,
                   gamma2, beta2, 128)
    agg = _sc_segsum(x3[0], x3[1], src_2d, dst_2d)
    x4 = _tc_graph(agg[0], agg[1], x3[0], x3[1], Wrel3, Wroot3, b3,
                   gamma3, beta3, 128)
    agg = _sc_segsum(x4[0], x4[1], src_2d, dst_2d)
    return _tc_graph(agg[0], agg[1], x4[0], x4[1], Wrel4, Wroot4, b4,
                     gamma4, beta4, 64)


# speedup vs baseline: 2.5846x; 1.0958x over previous
import functools

import jax
import jax.numpy as jnp
from jax import lax
from jax.experimental import pallas as pl
from jax.experimental.pallas import tpu as pltpu
from jax.experimental.pallas import tpu_sc as plsc

N = 10000
E = 320000
EPS = 1e-5

NC = 2
NS = 16
NW = NC * NS
CHUNK = 128
EPT = 10240
CPT = EPT // CHUNK
EP = NW * EPT
EPAD = 2 * CHUNK
NP = 10112
RPT = NP // NS
DCH = 128
DCPT = EPT // DCH
GRP = 16

def _stripe_chunks(step):
    out, off = [], 0
    while off < RPT:
        out.append((off, min(step, RPT - off)))
        off += step
    return tuple(out)


def _zero_vmem_2d(buf, nrows, ncols16):
    z = jnp.zeros((16,), jnp.float32)

    def body(i, c):
        for j in range(ncols16):
            buf[i, pl.ds(16 * j, 16)] = z
        return c

    lax.fori_loop(0, nrows, body, 0)


HC = 64
TCPT = EP // CHUNK // NS


def _segsum_body(h0_hbm, h1_hbm, src_hbm, dst_hbm, out0_hbm, out1_hbm,
                 hsh, acc, sidxb, didxb, rows0, rows1, sg0, sg1):
    cid = lax.axis_index("c")
    sid = lax.axis_index("s")
    rbase = sid * RPT

    @pl.when(cid == 0)
    def _stage0():
        pltpu.sync_copy(h0_hbm.at[pl.ds(rbase, RPT)],
                        hsh.at[pl.ds(rbase, RPT)])

    @pl.when(cid == 1)
    def _stage1():
        pltpu.sync_copy(h1_hbm.at[pl.ds(rbase, RPT)],
                        hsh.at[pl.ds(rbase, RPT)])

    _zero_vmem_2d(rows0, CHUNK, HC // 16)
    for off, n in _stripe_chunks(CHUNK):
        pltpu.sync_copy(rows0.at[pl.ds(0, n)], acc.at[pl.ds(rbase + off, n)])
    plsc.subcore_barrier()

    def body(j, c):
        crow = sid * TCPT + j * GRP
        pltpu.sync_copy(src_hbm.at[pl.ds(crow, GRP)], sidxb)
        pltpu.sync_copy(dst_hbm.at[pl.ds(crow, GRP)], didxb)
        bufs = (rows0, rows1)
        sems = (sg0, sg1)
        g = pltpu.async_copy(hsh.at[sidxb.at[0]], rows0, sg0)
        for k in range(GRP):
            nxt = None
            if k + 1 < GRP:
                nxt = pltpu.async_copy(hsh.at[sidxb.at[k + 1]],
                                       bufs[(k + 1) % 2], sems[(k + 1) % 2])
            g.wait()
            pltpu.sync_copy(bufs[k % 2], acc.at[didxb.at[k]], add=True)
            g = nxt
        return c

    lax.fori_loop(0, TCPT // GRP, body, 0)
    plsc.subcore_barrier()

    @pl.when(cid == 0)
    def _out0():
        pltpu.sync_copy(acc.at[pl.ds(rbase, RPT)],
                        out0_hbm.at[pl.ds(rbase, RPT)])

    @pl.when(cid == 1)
    def _out1():
        pltpu.sync_copy(acc.at[pl.ds(rbase, RPT)],
                        out1_hbm.at[pl.ds(rbase, RPT)])


def _sc_segsum(h0, h1, src_p, dst_p):
    mesh = plsc.VectorSubcoreMesh(core_axis_name="c", subcore_axis_name="s")
    return pl.kernel(
        _segsum_body,
        out_type=(jax.ShapeDtypeStruct((NP, HC), jnp.float32),
                  jax.ShapeDtypeStruct((NP, HC), jnp.float32)),
        mesh=mesh,
        scratch_types=[
            pltpu.VMEM_SHARED((NP, HC), jnp.float32),
            pltpu.VMEM_SHARED((NP, HC), jnp.float32),
            pltpu.VMEM((GRP, CHUNK), jnp.int32),
            pltpu.VMEM((GRP, CHUNK), jnp.int32),
            pltpu.VMEM((CHUNK, HC), jnp.float32),
            pltpu.VMEM((CHUNK, HC), jnp.float32),
            pltpu.SemaphoreType.DMA,
            pltpu.SemaphoreType.DMA,
        ],
    )(h0, h1, src_p, dst_p)


def _deg_body(dst_hbm, out_hbm, acc, didx, ones, zbuf):
    cid = lax.axis_index("c")
    sid = lax.axis_index("s")

    _zero_vmem_2d(zbuf, DCH, 1)
    ov = jnp.ones((16,), jnp.float32)

    def fill(i, c):
        ones[i, :] = ov
        return c

    lax.fori_loop(0, DCH, fill, 0)

    rbase = sid * RPT
    for off, n in _stripe_chunks(DCH):
        pltpu.sync_copy(zbuf.at[pl.ds(0, n)], acc.at[pl.ds(rbase + off, n)])
    plsc.subcore_barrier()

    ebase = (cid * NS + sid) * EPT

    def body(t, c):
        pltpu.sync_copy(dst_hbm.at[pl.ds(ebase + t * DCH, DCH)], didx)
        pltpu.sync_copy(ones, acc.at[didx], add=True)
        return c

    lax.fori_loop(0, DCPT, body, 0)
    plsc.subcore_barrier()
    pltpu.sync_copy(acc.at[pl.ds(rbase, RPT)],
                    out_hbm.at[cid, pl.ds(rbase, RPT)])


def _sc_deg(dst_p):
    mesh = plsc.VectorSubcoreMesh(core_axis_name="c", subcore_axis_name="s")
    return pl.kernel(
        _deg_body,
        out_type=jax.ShapeDtypeStruct((NC, NP, 16), jnp.float32),
        mesh=mesh,
        scratch_types=[
            pltpu.VMEM_SHARED((NP, 16), jnp.float32),
            pltpu.VMEM((DCH,), jnp.int32),
            pltpu.VMEM((DCH, 16), jnp.float32),
            pltpu.VMEM((DCH, 16), jnp.float32),
        ],
    )(dst_p)


def _batch_norm(h, gamma, beta):
    m = jnp.mean(h, axis=0, keepdims=True)
    hc = h - m
    v = jnp.mean(hc * hc, axis=0, keepdims=True)
    return hc * lax.rsqrt(v + EPS) * gamma[None, :] + beta[None, :]


def _dinv_from_parts(degp_ref):
    degp = degp_ref[...]
    deg = degp[0, :, 0:1] + degp[1, :, 0:1] + 1.0
    return lax.rsqrt(deg)


def _store_split(ref0, ref1, mat):
    zpad = jnp.zeros((NP - N, HC), jnp.float32)
    ref0[pl.ds(0, N), :] = mat[:, 0:HC]
    ref0[pl.ds(N, NP - N), :] = zpad
    ref1[pl.ds(0, N), :] = mat[:, HC:128]
    ref1[pl.ds(N, NP - N), :] = zpad


def _tc1_body(x_ref, w_ref, g_ref, b_ref, degp_ref, hs0_ref, hs1_ref):
    x = x_ref[...]
    xn = _batch_norm(x, g_ref[...], b_ref[...])
    h1 = jnp.dot(xn, w_ref[...], preferred_element_type=jnp.float32)
    dinv = _dinv_from_parts(degp_ref)
    _store_split(hs0_ref, hs1_ref, h1 * dinv[0:N])


def _tc1(x, w1, g0, b0, degp):
    return pl.pallas_call(
        _tc1_body,
        out_shape=(jax.ShapeDtypeStruct((NP, HC), jnp.float32),
                   jax.ShapeDtypeStruct((NP, HC), jnp.float32)),
    )(x, w1, g0, b0, degp)


def _tc2_body(seg0_ref, seg1_ref, hs0_ref, hs1_ref, degp_ref, b_ref,
              g1_ref, be1_ref, out0_ref, out1_ref):
    dinv = _dinv_from_parts(degp_ref)[0:N]
    seg = jnp.concatenate(
        [seg0_ref[pl.ds(0, N), :], seg1_ref[pl.ds(0, N), :]], axis=1)
    hs = jnp.concatenate(
        [hs0_ref[pl.ds(0, N), :], hs1_ref[pl.ds(0, N), :]], axis=1)
    h = jax.nn.relu(dinv * (seg + hs) + b_ref[...][None, :])
    _store_split(out0_ref, out1_ref,
                 _batch_norm(h, g1_ref[...], be1_ref[...]))


def _tc2(seg0, seg1, hs0, hs1, degp, b1, g1, be1):
    return pl.pallas_call(
        _tc2_body,
        out_shape=(jax.ShapeDtypeStruct((NP, HC), jnp.float32),
                   jax.ShapeDtypeStruct((NP, HC), jnp.float32)),
    )(seg0, seg1, hs0, hs1, degp, b1, g1, be1)


def _tc_graph_body(agg0_ref, agg1_ref, x0_ref, x1_ref, wrel_ref, wroot_ref,
                   b_ref, g_ref, be_ref, *out_refs, hout):
    wrel = wrel_ref[...]
    wroot = wroot_ref[...]
    y = (jnp.dot(agg0_ref[pl.ds(0, N), :], wrel[0:HC],
                 preferred_element_type=jnp.float32)
         + jnp.dot(agg1_ref[pl.ds(0, N), :], wrel[HC:128],
                   preferred_element_type=jnp.float32)
         + jnp.dot(x0_ref[pl.ds(0, N), :], wroot[0:HC],
                   preferred_element_type=jnp.float32)
         + jnp.dot(x1_ref[pl.ds(0, N), :], wroot[HC:128],
                   preferred_element_type=jnp.float32)
         + b_ref[...][None, :])
    h = _batch_norm(jax.nn.relu(y), g_ref[...], be_ref[...])
    if hout == 128:
        _store_split(out_refs[0], out_refs[1], h)
    else:
        out_refs[0][...] = h


def _tc_graph(agg0, agg1, x0, x1, wrel, wroot, b, g, be, hout):
    if hout == 128:
        out_shape = (jax.ShapeDtypeStruct((NP, HC), jnp.float32),
                     jax.ShapeDtypeStruct((NP, HC), jnp.float32))
    else:
        out_shape = jax.ShapeDtypeStruct((N, hout), jnp.float32)
    return pl.pallas_call(
        functools.partial(_tc_graph_body, hout=hout),
        out_shape=out_shape,
    )(agg0, agg1, x0, x1, wrel, wroot, b, g, be)


def kernel(x, edge_index, gamma0, beta0, W1, b1, gamma1, beta1, Wrel2,
           Wroot2, b2, gamma2, beta2, Wrel3, Wroot3, b3, gamma3, beta3,
           Wrel4, Wroot4, b4, gamma4, beta4):
    pad = jnp.full((EP - E,), N, dtype=jnp.int32)
    src_p = jnp.concatenate([edge_index[0], pad])
    dst_p = jnp.concatenate([edge_index[1], pad])
    src_2d = src_p.reshape(EP // CHUNK, CHUNK)
    dst_2d = dst_p.reshape(EP // CHUNK, CHUNK)

    degp = _sc_deg(dst_p)
    hs0, hs1 = _tc1(x, W1, gamma0, beta0, degp)
    seg0, seg1 = _sc_segsum(hs0, hs1, src_2d, dst_2d)
    x2 = _tc2(seg0, seg1, hs0, hs1, degp, b1, gamma1, beta1)
    agg = _sc_segsum(x2[0], x2[1], src_2d, dst_2d)
    x3 = _tc_graph(agg[0], agg[1], x2[0], x2[1], Wrel2, Wroot2, b2,
                   gamma2, beta2, 128)
    agg = _sc_segsum(x3[0], x3[1], src_2d, dst_2d)
    x4 = _tc_graph(agg[0], agg[1], x3[0], x3[1], Wrel3, Wroot3, b3,
                   gamma3, beta3, 128)
    agg = _sc_segsum(x4[0], x4[1], src_2d, dst_2d)
    return _tc_graph(agg[0], agg[1], x4[0], x4[1], Wrel4, Wroot4, b4,
                     gamma4, beta4, 64)


# speedup vs baseline: 2.5863x; 1.0007x over previous
import functools

import jax
import jax.numpy as jnp
from jax import lax
from jax.experimental import pallas as pl
from jax.experimental.pallas import tpu as pltpu
from jax.experimental.pallas import tpu_sc as plsc

N = 10000
E = 320000
EPS = 1e-5

NC = 2
NS = 16
NW = NC * NS
CHUNK = 128
EPT = 10240
CPT = EPT // CHUNK
EP = NW * EPT
EPAD = 2 * CHUNK
NP = 10112
RPT = NP // NS
DCH = 128
DCPT = EPT // DCH
GRP = 16

def _stripe_chunks(step):
    out, off = [], 0
    while off < RPT:
        out.append((off, min(step, RPT - off)))
        off += step
    return tuple(out)


def _zero_vmem_2d(buf, nrows, ncols16):
    z = jnp.zeros((16,), jnp.float32)

    def body(i, c):
        for j in range(ncols16):
            buf[i, pl.ds(16 * j, 16)] = z
        return c

    lax.fori_loop(0, nrows, body, 0)


HC = 64
TCPT = EP // CHUNK // NS


def _segsum_body(h0_hbm, h1_hbm, src_hbm, dst_hbm, out0_hbm, out1_hbm,
                 hsh, acc, sidxb, didxb, rows0, rows1, sg0, sg1):
    cid = lax.axis_index("c")
    sid = lax.axis_index("s")
    rbase = sid * RPT

    @pl.when(cid == 0)
    def _stage0():
        pltpu.sync_copy(h0_hbm.at[pl.ds(rbase, RPT)],
                        hsh.at[pl.ds(rbase, RPT)])

    @pl.when(cid == 1)
    def _stage1():
        pltpu.sync_copy(h1_hbm.at[pl.ds(rbase, RPT)],
                        hsh.at[pl.ds(rbase, RPT)])

    _zero_vmem_2d(rows0, CHUNK, HC // 16)
    for off, n in _stripe_chunks(CHUNK):
        pltpu.sync_copy(rows0.at[pl.ds(0, n)], acc.at[pl.ds(rbase + off, n)])
    plsc.subcore_barrier()

    def body(j, c):
        crow = sid * TCPT + j * GRP
        pltpu.sync_copy(src_hbm.at[pl.ds(crow, GRP)], sidxb)
        pltpu.sync_copy(dst_hbm.at[pl.ds(crow, GRP)], didxb)
        bufs = (rows0, rows1)
        sems = (sg0, sg1)
        g = pltpu.async_copy(hsh.at[sidxb.at[0]], rows0, sg0)
        for k in range(GRP):
            nxt = None
            if k + 1 < GRP:
                nxt = pltpu.async_copy(hsh.at[sidxb.at[k + 1]],
                                       bufs[(k + 1) % 2], sems[(k + 1) % 2])
            g.wait()
            pltpu.sync_copy(bufs[k % 2], acc.at[didxb.at[k]], add=True)
            g = nxt
        return c

    lax.fori_loop(0, TCPT // GRP, body, 0)
    plsc.subcore_barrier()

    @pl.when(cid == 0)
    def _out0():
        pltpu.sync_copy(acc.at[pl.ds(rbase, RPT)],
                        out0_hbm.at[pl.ds(rbase, RPT)])

    @pl.when(cid == 1)
    def _out1():
        pltpu.sync_copy(acc.at[pl.ds(rbase, RPT)],
                        out1_hbm.at[pl.ds(rbase, RPT)])


def _sc_segsum(h0, h1, src_p, dst_p):
    mesh = plsc.VectorSubcoreMesh(core_axis_name="c", subcore_axis_name="s")
    return pl.kernel(
        _segsum_body,
        out_type=(jax.ShapeDtypeStruct((NP, HC), jnp.float32),
                  jax.ShapeDtypeStruct((NP, HC), jnp.float32)),
        mesh=mesh,
        scratch_types=[
            pltpu.VMEM_SHARED((NP, HC), jnp.float32),
            pltpu.VMEM_SHARED((NP, HC), jnp.float32),
            pltpu.VMEM((GRP, CHUNK), jnp.int32),
            pltpu.VMEM((GRP, CHUNK), jnp.int32),
            pltpu.VMEM((CHUNK, HC), jnp.float32),
            pltpu.VMEM((CHUNK, HC), jnp.float32),
            pltpu.SemaphoreType.DMA,
            pltpu.SemaphoreType.DMA,
        ],
    )(h0, h1, src_p, dst_p)


def _deg_body(dst_hbm, out_hbm, acc, didx, ones, zbuf):
    cid = lax.axis_index("c")
    sid = lax.axis_index("s")

    _zero_vmem_2d(zbuf, DCH, 1)
    ov = jnp.ones((16,), jnp.float32)

    def fill(i, c):
        ones[i, :] = ov
        return c

    lax.fori_loop(0, DCH, fill, 0)

    rbase = sid * RPT
    for off, n in _stripe_chunks(DCH):
        pltpu.sync_copy(zbuf.at[pl.ds(0, n)], acc.at[pl.ds(rbase + off, n)])
    plsc.subcore_barrier()

    ebase = (cid * NS + sid) * EPT

    def body(t, c):
        pltpu.sync_copy(dst_hbm.at[pl.ds(ebase + t * DCH, DCH)], didx)
        pltpu.sync_copy(ones, acc.at[didx], add=True)
        return c

    lax.fori_loop(0, DCPT, body, 0)
    plsc.subcore_barrier()
    pltpu.sync_copy(acc.at[pl.ds(rbase, RPT)],
                    out_hbm.at[cid, pl.ds(rbase, RPT)])


def _sc_deg(dst_p):
    mesh = plsc.VectorSubcoreMesh(core_axis_name="c", subcore_axis_name="s")
    return pl.kernel(
        _deg_body,
        out_type=jax.ShapeDtypeStruct((NC, NP, 16), jnp.float32),
        mesh=mesh,
        scratch_types=[
            pltpu.VMEM_SHARED((NP, 16), jnp.float32),
            pltpu.VMEM((DCH,), jnp.int32),
            pltpu.VMEM((DCH, 16), jnp.float32),
            pltpu.VMEM((DCH, 16), jnp.float32),
        ],
    )(dst_p)


def _batch_norm(h, gamma, beta):
    m = jnp.mean(h, axis=0, keepdims=True)
    hc = h - m
    v = jnp.mean(hc * hc, axis=0, keepdims=True)
    return hc * lax.rsqrt(v + EPS) * gamma[None, :] + beta[None, :]


def _dinv_from_parts(degp_ref):
    degp = degp_ref[...]
    deg = degp[0, :, 0:1] + degp[1, :, 0:1] + 1.0
    return lax.rsqrt(deg)


def _store_split(ref0, ref1, mat):
    zpad = jnp.zeros((NP - N, HC), jnp.float32)
    ref0[pl.ds(0, N), :] = mat[:, 0:HC]
    ref0[pl.ds(N, NP - N), :] = zpad
    ref1[pl.ds(0, N), :] = mat[:, HC:128]
    ref1[pl.ds(N, NP - N), :] = zpad


def _tc1a_body(x_ref, w_ref, g_ref, b_ref, h1_ref):
    x = x_ref[...]
    xn = _batch_norm(x, g_ref[...], b_ref[...])
    h1_ref[...] = jnp.dot(xn, w_ref[...], preferred_element_type=jnp.float32)


def _tc1a(x, w1, g0, b0):
    return pl.pallas_call(
        _tc1a_body,
        out_shape=jax.ShapeDtypeStruct((N, 128), jnp.float32),
    )(x, w1, g0, b0)


def _tc1b_body(h1_ref, degp_ref, hs0_ref, hs1_ref):
    dinv = _dinv_from_parts(degp_ref)
    _store_split(hs0_ref, hs1_ref, h1_ref[...] * dinv[0:N])


def _tc1b(h1, degp):
    return pl.pallas_call(
        _tc1b_body,
        out_shape=(jax.ShapeDtypeStruct((NP, HC), jnp.float32),
                   jax.ShapeDtypeStruct((NP, HC), jnp.float32)),
    )(h1, degp)


def _tc2_body(seg0_ref, seg1_ref, hs0_ref, hs1_ref, degp_ref, b_ref,
              g1_ref, be1_ref, out0_ref, out1_ref):
    dinv = _dinv_from_parts(degp_ref)[0:N]
    seg = jnp.concatenate(
        [seg0_ref[pl.ds(0, N), :], seg1_ref[pl.ds(0, N), :]], axis=1)
    hs = jnp.concatenate(
        [hs0_ref[pl.ds(0, N), :], hs1_ref[pl.ds(0, N), :]], axis=1)
    h = jax.nn.relu(dinv * (seg + hs) + b_ref[...][None, :])
    _store_split(out0_ref, out1_ref,
                 _batch_norm(h, g1_ref[...], be1_ref[...]))


def _tc2(seg0, seg1, hs0, hs1, degp, b1, g1, be1):
    return pl.pallas_call(
        _tc2_body,
        out_shape=(jax.ShapeDtypeStruct((NP, HC), jnp.float32),
                   jax.ShapeDtypeStruct((NP, HC), jnp.float32)),
    )(seg0, seg1, hs0, hs1, degp, b1, g1, be1)


def _tc_graph_body(agg0_ref, agg1_ref, x0_ref, x1_ref, wrel_ref, wroot_ref,
                   b_ref, g_ref, be_ref, *out_refs, hout):
    wrel = wrel_ref[...]
    wroot = wroot_ref[...]
    y = (jnp.dot(agg0_ref[pl.ds(0, N), :], wrel[0:HC],
                 preferred_element_type=jnp.float32)
         + jnp.dot(agg1_ref[pl.ds(0, N), :], wrel[HC:128],
                   preferred_element_type=jnp.float32)
         + jnp.dot(x0_ref[pl.ds(0, N), :], wroot[0:HC],
                   preferred_element_type=jnp.float32)
         + jnp.dot(x1_ref[pl.ds(0, N), :], wroot[HC:128],
                   preferred_element_type=jnp.float32)
         + b_ref[...][None, :])
    h = _batch_norm(jax.nn.relu(y), g_ref[...], be_ref[...])
    if hout == 128:
        _store_split(out_refs[0], out_refs[1], h)
    else:
        out_refs[0][...] = h


def _tc_graph(agg0, agg1, x0, x1, wrel, wroot, b, g, be, hout):
    if hout == 128:
        out_shape = (jax.ShapeDtypeStruct((NP, HC), jnp.float32),
                     jax.ShapeDtypeStruct((NP, HC), jnp.float32))
    else:
        out_shape = jax.ShapeDtypeStruct((N, hout), jnp.float32)
    return pl.pallas_call(
        functools.partial(_tc_graph_body, hout=hout),
        out_shape=out_shape,
    )(agg0, agg1, x0, x1, wrel, wroot, b, g, be)


def kernel(x, edge_index, gamma0, beta0, W1, b1, gamma1, beta1, Wrel2,
           Wroot2, b2, gamma2, beta2, Wrel3, Wroot3, b3, gamma3, beta3,
           Wrel4, Wroot4, b4, gamma4, beta4):
    pad = jnp.full((EP - E,), N, dtype=jnp.int32)
    src_p = jnp.concatenate([edge_index[0], pad])
    dst_p = jnp.concatenate([edge_index[1], pad])
    src_2d = src_p.reshape(EP // CHUNK, CHUNK)
    dst_2d = dst_p.reshape(EP // CHUNK, CHUNK)

    degp = _sc_deg(dst_p)
    h1 = _tc1a(x, W1, gamma0, beta0)
    hs0, hs1 = _tc1b(h1, degp)
    seg0, seg1 = _sc_segsum(hs0, hs1, src_2d, dst_2d)
    x2 = _tc2(seg0, seg1, hs0, hs1, degp, b1, gamma1, beta1)
    agg = _sc_segsum(x2[0], x2[1], src_2d, dst_2d)
    x3 = _tc_graph(agg[0], agg[1], x2[0], x2[1], Wrel2, Wroot2, b2,
                   gamma2, beta2, 128)
    agg = _sc_segsum(x3[0], x3[1], src_2d, dst_2d)
    x4 = _tc_graph(agg[0], agg[1], x3[0], x3[1], Wrel3, Wroot3, b3,
                   gamma3, beta3, 128)
    agg = _sc_segsum(x4[0], x4[1], src_2d, dst_2d)
    return _tc_graph(agg[0], agg[1], x4[0], x4[1], Wrel4, Wroot4, b4,
                     gamma4, beta4, 64)
